# baseline (speedup 1.0000x reference)
import functools

import jax
import jax.numpy as jnp
from jax import lax
from jax.experimental import pallas as pl
from jax.experimental.pallas import tpu as pltpu
from jax.experimental.pallas import tpu_sc as plsc

N = 10000
E = 160000
DIN = 128
DH = 128
H = 8
D = 16
HP = 16
DOUT = 8
WOUT = DH + HP

BLK = 2000
K = 112
EPW = 5152
EPAD = EPW * 32
NCH = EPW // K
RPT = N // 16
ZR = 5


def _tc1_body(xS, xE, WS, WE, bS, bE, A0, A1, A2, A3,
              xts_o, xte_o, t0_o, t2_o, mx_o):
    xs = jnp.dot(xS[...], WS[...], preferred_element_type=jnp.float32) + bS[...]
    xe = jnp.dot(xE[...], WE[...], preferred_element_type=jnp.float32) + bE[...]
    t0 = jnp.dot(xs, A0[...], preferred_element_type=jnp.float32)
    t1 = jnp.dot(xs, A1[...], preferred_element_type=jnp.float32)
    t2 = jnp.dot(xs, A2[...], preferred_element_type=jnp.float32)
    t3 = jnp.dot(xe, A3[...], preferred_element_type=jnp.float32)
    xts_o[...] = jnp.concatenate([xs, t1], axis=1)
    xte_o[...] = jnp.concatenate([xe, t3], axis=1)
    t0_o[...] = t0
    t2_o[...] = t2
    m = jnp.concatenate(
        [jnp.max(t0, axis=0, keepdims=True), jnp.max(t1, axis=0, keepdims=True),
         jnp.max(t2, axis=0, keepdims=True), jnp.max(t3, axis=0, keepdims=True)],
        axis=1)
    mb = jnp.broadcast_to(m, (8, 64))

    @pl.when(pl.program_id(0) == 0)
    def _():
        mx_o[...] = mb

    @pl.when(pl.program_id(0) != 0)
    def _():
        mx_o[...] = jnp.maximum(mx_o[...], mb)


_tbl = jax.ShapeDtypeStruct((N, HP), jnp.float32)
_tc1 = pl.pallas_call(
    _tc1_body,
    grid=(N // BLK,),
    in_specs=[
        pl.BlockSpec((BLK, DIN), lambda i: (i, 0)),
        pl.BlockSpec((BLK, DIN), lambda i: (i, 0)),
        pl.BlockSpec((DIN, DH), lambda i: (0, 0)),
        pl.BlockSpec((DIN, DH), lambda i: (0, 0)),
        pl.BlockSpec((1, DH), lambda i: (0, 0)),
        pl.BlockSpec((1, DH), lambda i: (0, 0)),
        pl.BlockSpec((DH, HP), lambda i: (0, 0)),
        pl.BlockSpec((DH, HP), lambda i: (0, 0)),
        pl.BlockSpec((DH, HP), lambda i: (0, 0)),
        pl.BlockSpec((DH, HP), lambda i: (0, 0)),
    ],
    out_specs=[
        pl.BlockSpec((BLK, WOUT), lambda i: (i, 0)),
        pl.BlockSpec((BLK, WOUT), lambda i: (i, 0)),
        pl.BlockSpec((BLK, HP), lambda i: (i, 0)),
        pl.BlockSpec((BLK, HP), lambda i: (i, 0)),
        pl.BlockSpec((8, 64), lambda i: (0, 0)),
    ],
    out_shape=[
        jax.ShapeDtypeStruct((N, WOUT), jnp.float32),
        jax.ShapeDtypeStruct((N, WOUT), jnp.float32),
        _tbl, _tbl,
        jax.ShapeDtypeStruct((8, 64), jnp.float32),
    ],
)


def _sc_body(src1_h, dst1_h, xt1_h, ad1_h, c1_h,
             src2_h, dst2_h, xt2_h, ad2_h, c2_h,
             out1_h, out2_h,
             sidx0, didx0, sidx1, didx1, dsc0, dsc1,
             adr0, xm0, adr1, xm1,
             zbuf, c_v, acc, semi0, semi1, semg0, semg1, sems):
    sidx = (sidx0, sidx1)
    didx = (didx0, didx1)
    dsc = (dsc0, dsc1)
    adr = (adr0, adr1)
    xm = (xm0, xm1)
    semi = (semi0, semi1)
    semg = (semg0, semg1)

    cid = lax.axis_index("c")
    sid = lax.axis_index("s")

    for i in range(ZR):
        for j in range(WOUT // 16):
            zbuf[i, pl.ds(j * 16, 16)] = jnp.zeros((16,), jnp.float32)

    gw = cid * 16 + sid
    ebase = gw * EPW

    def run_type(src_h, dst_h, xt_h, ad_h, c_h, out_h):
        def zloop(t, carry):
            pltpu.async_copy(zbuf, acc.at[pl.ds(sid * RPT + t * ZR, ZR)],
                             semg0)
            return carry

        lax.fori_loop(0, RPT // ZR, zloop, 0)

        def zdrain(t, carry):
            pltpu.make_async_copy(zbuf, acc.at[pl.ds(sid * RPT, ZR)],
                                  semg0).wait()
            return carry

        lax.fori_loop(0, RPT // ZR, zdrain, 0)
        pltpu.sync_copy(c_h, c_v)
        plsc.subcore_barrier()

        cv = c_v[...]

        def stage_gathers(eb1, q):
            pltpu.async_copy(ad_h.at[didx[q]], adr[q], semg[q])
            pltpu.async_copy(xt_h.at[sidx[q]], xm[q], semg[q])
            pltpu.async_copy(dst_h.at[pl.ds(eb1, K)], dsc[q], semg[q])

        def drain_gathers(b):
            pltpu.make_async_copy(ad_h.at[didx[b]], adr[b], semg[b]).wait()
            pltpu.make_async_copy(xt_h.at[sidx[b]], xm[b], semg[b]).wait()
            pltpu.make_async_copy(dst_h.at[pl.ds(ebase, K)], dsc[b],
                                  semg[b]).wait()

        def drain_scatter(b):
            pltpu.make_async_copy(xm[b], acc.at[dsc[b]], sems).wait()

        def drain_idx(q):
            pltpu.make_async_copy(src_h.at[pl.ds(ebase, K)], sidx[q],
                                  semi[q]).wait()
            pltpu.make_async_copy(dst_h.at[pl.ds(ebase, K)], didx[q],
                                  semi[q]).wait()

        for b in range(2):
            pltpu.sync_copy(src_h.at[pl.ds(ebase + b * K, K)], sidx[b])
            pltpu.sync_copy(dst_h.at[pl.ds(ebase + b * K, K)], didx[b])
            stage_gathers(ebase + b * K, b)

        def emit(ci, b):
            q = 1 - b
            drain_gathers(b)

            @pl.when(ci < NCH - 2)
            def _():
                eb2 = ebase + (ci + 2) * K
                pltpu.async_copy(src_h.at[pl.ds(eb2, K)], sidx[b], semi[b])
                pltpu.async_copy(dst_h.at[pl.ds(eb2, K)], didx[b], semi[b])

            @pl.when(ci >= 1)
            def _():
                drain_scatter(b)

            @pl.when((ci >= 1) & (ci < NCH - 1))
            def _():
                drain_idx(q)
                stage_gathers(ebase + (ci + 1) * K, q)

            eb = ebase + ci * K

            @plsc.parallel_loop(0, K, step=1, unroll=6)
            def _grp(k0):
                av = xm[b][k0, pl.ds(DH, 16)] + adr[b][k0]
                a = jnp.where(av >= 0.0, av, 0.2 * av)
                ev = jnp.exp(a - cv)
                ev = jnp.where(eb + k0 >= E, 0.0, ev)
                xm[b][k0, pl.ds(DH, 16)] = ev
                for h in range(H):
                    bc = ev.at[jnp.full((16,), h, jnp.int32)].get(
                        mode="promise_in_bounds")
                    xm[b][k0, pl.ds(h * 16, 16)] = (
                        xm[b][k0, pl.ds(h * 16, 16)] * bc)

            pltpu.async_copy(xm[b], acc.at[dsc[b]], sems, add=True)

        def pair(it, carry):
            emit(2 * it, 0)
            emit(2 * it + 1, 1)
            return carry

        lax.fori_loop(0, NCH // 2, pair, 0)
        drain_scatter(1)
        plsc.subcore_barrier()
        pltpu.sync_copy(acc.at[pl.ds(sid * RPT, RPT)],
                        out_h.at[cid, pl.ds(sid * RPT, RPT)])

    run_type(src1_h, dst1_h, xt1_h, ad1_h, c1_h, out1_h)
    run_type(src2_h, dst2_h, xt2_h, ad2_h, c2_h, out2_h)


_scratch_f32 = lambda shape: pltpu.VMEM(shape, jnp.float32)
_out_p = jax.ShapeDtypeStruct((2, N, WOUT), jnp.float32)
_sc_edge = functools.partial(
    pl.kernel,
    mesh=plsc.VectorSubcoreMesh(core_axis_name="c", subcore_axis_name="s"),
    compiler_params=pltpu.CompilerParams(use_tc_tiling_on_sc=False),
    out_type=[_out_p, _out_p],
    scratch_types=[
        pltpu.VMEM((K,), jnp.int32),
        pltpu.VMEM((K,), jnp.int32),
        pltpu.VMEM((K,), jnp.int32),
        pltpu.VMEM((K,), jnp.int32),
        pltpu.VMEM((K,), jnp.int32),
        pltpu.VMEM((K,), jnp.int32),
        _scratch_f32((K, HP)),
        _scratch_f32((K, WOUT)),
        _scratch_f32((K, HP)),
        _scratch_f32((K, WOUT)),
        _scratch_f32((ZR, WOUT)),
        _scratch_f32((16,)),
        pltpu.VMEM_SHARED((N, WOUT), jnp.float32),
        pltpu.SemaphoreType.DMA,
        pltpu.SemaphoreType.DMA,
        pltpu.SemaphoreType.DMA,
        pltpu.SemaphoreType.DMA,
        pltpu.SemaphoreType.DMA,
    ],
)(_sc_body)


def _tc2_body(p1, p2, Wk, bk, q2, Bx, Wo, bo, y_o, o1s, o2s, sacc):
    ph = pl.program_id(0)
    j = pl.program_id(1)

    @pl.when(ph == 0)
    def _():
        def one(p):
            ps = p[0] + p[1]
            denx = jnp.dot(ps[:, DH:WOUT], Bx[...],
                           preferred_element_type=jnp.float32)
            return jnp.maximum(ps[:, 0:DH] / (denx + 1e-16), 0.0)

        o1 = one(p1[...])
        o2 = one(p2[...])
        o1s[pl.ds(j * BLK, BLK), :] = o1
        o2s[pl.ds(j * BLK, BLK), :] = o2
        kp1 = jnp.tanh(jnp.dot(o1, Wk[...],
                               preferred_element_type=jnp.float32) + bk[...])
        kp2 = jnp.tanh(jnp.dot(o2, Wk[...],
                               preferred_element_type=jnp.float32) + bk[...])
        s1 = jnp.sum(kp1 * q2[...]) * (1.0 / N)
        s2 = jnp.sum(kp2 * q2[...]) * (1.0 / N)
        col = lax.broadcasted_iota(jnp.int32, (1, 8), 1)
        part = jnp.where(col == 0, s1, 0.0) + jnp.where(col == 1, s2, 0.0)

        @pl.when(j == 0)
        def _():
            sacc[...] = part

        @pl.when(j != 0)
        def _():
            sacc[...] = sacc[...] + part

    @pl.when(ph == 1)
    def _():
        sv = sacc[...]
        s1 = sv[0:1, 0:1]
        s2 = sv[0:1, 1:2]
        m = jnp.maximum(s1, s2)
        e1 = jnp.exp(s1 - m)
        e2 = jnp.exp(s2 - m)
        a1 = e1 / (e1 + e2)
        a2 = e2 / (e1 + e2)
        fused = (o1s[pl.ds(j * BLK, BLK), :] * a1 +
                 o2s[pl.ds(j * BLK, BLK), :] * a2)
        y_o[...] = jnp.dot(fused, Wo[...],
                           preferred_element_type=jnp.float32) + bo[...]


_tc2 = pl.pallas_call(
    _tc2_body,
    grid=(2, N // BLK),
    in_specs=[
        pl.BlockSpec((2, BLK, WOUT), lambda ph, j: (0, j * (1 - ph), 0)),
        pl.BlockSpec((2, BLK, WOUT), lambda ph, j: (0, j * (1 - ph), 0)),
        pl.BlockSpec((DH, DH), lambda ph, j: (0, 0)),
        pl.BlockSpec((1, DH), lambda ph, j: (0, 0)),
        pl.BlockSpec((1, DH), lambda ph, j: (0, 0)),
        pl.BlockSpec((HP, DH), lambda ph, j: (0, 0)),
        pl.BlockSpec((DH, DOUT), lambda ph, j: (0, 0)),
        pl.BlockSpec((1, DOUT), lambda ph, j: (0, 0)),
    ],
    out_specs=pl.BlockSpec((BLK, DOUT), lambda ph, j: (j, 0)),
    out_shape=jax.ShapeDtypeStruct((N, DOUT), jnp.float32),
    scratch_shapes=[
        pltpu.VMEM((N, DH), jnp.float32),
        pltpu.VMEM((N, DH), jnp.float32),
        pltpu.VMEM((1, 8), jnp.float32),
    ],
)


def _mkA(att):
    rows = jnp.arange(DH)[:, None]
    cols = jnp.arange(HP)[None, :]
    return jnp.where(cols == rows // D, att.reshape(-1)[:, None], 0.0)


def _leaky(v):
    return jnp.where(v >= 0.0, v, 0.2 * v)


def kernel(x_SUBJECT, x_ELECTRODE, edge_index_e2s, edge_index_s2s,
           W_proj_SUBJECT, b_proj_SUBJECT, W_proj_ELECTRODE, b_proj_ELECTRODE,
           att_src_e2s, att_dst_e2s, att_src_s2s, att_dst_s2s,
           W_k, b_k, q, W_out, b_out):
    A0 = _mkA(att_dst_e2s)
    A1 = _mkA(att_src_s2s)
    A2 = _mkA(att_dst_s2s)
    A3 = _mkA(att_src_e2s)

    xts, xte, t0, t2, mx = _tc1(
        x_SUBJECT, x_ELECTRODE, W_proj_SUBJECT, W_proj_ELECTRODE,
        b_proj_SUBJECT.reshape(1, DH), b_proj_ELECTRODE.reshape(1, DH),
        A0, A1, A2, A3)

    c_e2s = _leaky(mx[0, 48:64] + mx[0, 0:16])
    c_s2s = _leaky(mx[0, 16:32] + mx[0, 32:48])

    padv = jnp.broadcast_to((jnp.arange(EPAD - E, dtype=jnp.int32) % N)[None],
                            (2, EPAD - E))
    ei1 = jnp.concatenate([edge_index_e2s, padv], axis=1)
    ei2 = jnp.concatenate([edge_index_s2s, padv], axis=1)
    p1, p2 = _sc_edge(ei1[0], ei1[1], xte, t0, c_e2s,
                      ei2[0], ei2[1], xts, t2, c_s2s)

    rr = jnp.arange(HP)[:, None]
    cc = jnp.arange(DH)[None, :]
    Bx = jnp.where(rr == cc // D, 1.0, 0.0).astype(jnp.float32)

    return _tc2(p1, p2, W_k, b_k.reshape(1, DH), q.reshape(1, DH), Bx,
                W_out, b_out.reshape(1, DOUT))


# speedup vs baseline: 1.3123x; 1.3123x over previous
import functools

import jax
import jax.numpy as jnp
from jax import lax
from jax.experimental import pallas as pl
from jax.experimental.pallas import tpu as pltpu
from jax.experimental.pallas import tpu_sc as plsc

N = 10000
E = 160000
DIN = 128
DH = 128
H = 8
D = 16
HP = 16
DOUT = 8
WOUT = DH + HP

BLK = 2000
K = 112
EPW = 5152
EPAD = EPW * 32
NCH = EPW // K
RPT = N // 16
ZR = 5


def _tc1_body(xS, xE, WS, WE, bS, bE, A0, A1, A2, A3,
              xts_o, xte_o, t0_o, t2_o, mx_o):
    xs = jnp.dot(xS[...], WS[...], preferred_element_type=jnp.float32) + bS[...]
    xe = jnp.dot(xE[...], WE[...], preferred_element_type=jnp.float32) + bE[...]
    t0 = jnp.dot(xs, A0[...], preferred_element_type=jnp.float32)
    t1 = jnp.dot(xs, A1[...], preferred_element_type=jnp.float32)
    t2 = jnp.dot(xs, A2[...], preferred_element_type=jnp.float32)
    t3 = jnp.dot(xe, A3[...], preferred_element_type=jnp.float32)
    xts_o[...] = jnp.concatenate([xs, t1], axis=1)
    xte_o[...] = jnp.concatenate([xe, t3], axis=1)
    t0_o[...] = t0
    t2_o[...] = t2
    m = jnp.concatenate(
        [jnp.max(t0, axis=0, keepdims=True), jnp.max(t1, axis=0, keepdims=True),
         jnp.max(t2, axis=0, keepdims=True), jnp.max(t3, axis=0, keepdims=True)],
        axis=1)
    mb = jnp.broadcast_to(m, (8, 64))

    @pl.when(pl.program_id(0) == 0)
    def _():
        mx_o[...] = mb

    @pl.when(pl.program_id(0) != 0)
    def _():
        mx_o[...] = jnp.maximum(mx_o[...], mb)


_tbl = jax.ShapeDtypeStruct((N, HP), jnp.float32)
_tc1 = pl.pallas_call(
    _tc1_body,
    grid=(N // BLK,),
    in_specs=[
        pl.BlockSpec((BLK, DIN), lambda i: (i, 0)),
        pl.BlockSpec((BLK, DIN), lambda i: (i, 0)),
        pl.BlockSpec((DIN, DH), lambda i: (0, 0)),
        pl.BlockSpec((DIN, DH), lambda i: (0, 0)),
        pl.BlockSpec((1, DH), lambda i: (0, 0)),
        pl.BlockSpec((1, DH), lambda i: (0, 0)),
        pl.BlockSpec((DH, HP), lambda i: (0, 0)),
        pl.BlockSpec((DH, HP), lambda i: (0, 0)),
        pl.BlockSpec((DH, HP), lambda i: (0, 0)),
        pl.BlockSpec((DH, HP), lambda i: (0, 0)),
    ],
    out_specs=[
        pl.BlockSpec((BLK, WOUT), lambda i: (i, 0)),
        pl.BlockSpec((BLK, WOUT), lambda i: (i, 0)),
        pl.BlockSpec((BLK, HP), lambda i: (i, 0)),
        pl.BlockSpec((BLK, HP), lambda i: (i, 0)),
        pl.BlockSpec((8, 64), lambda i: (0, 0)),
    ],
    out_shape=[
        jax.ShapeDtypeStruct((N, WOUT), jnp.float32),
        jax.ShapeDtypeStruct((N, WOUT), jnp.float32),
        _tbl, _tbl,
        jax.ShapeDtypeStruct((8, 64), jnp.float32),
    ],
)


def _sc_body(src1_h, dst1_h, xt1_h, ad1_h, c1_h,
             src2_h, dst2_h, xt2_h, ad2_h, c2_h,
             out1_h, out2_h,
             sidx0, didx0, sidx1, didx1, dsc0, dsc1,
             adr0, xm0, adr1, xm1,
             zbuf, c_v, acc, semi0, semi1, semg0, semg1, sems):
    sidx = (sidx0, sidx1)
    didx = (didx0, didx1)
    dsc = (dsc0, dsc1)
    adr = (adr0, adr1)
    xm = (xm0, xm1)
    semi = (semi0, semi1)
    semg = (semg0, semg1)

    cid = lax.axis_index("c")
    sid = lax.axis_index("s")

    for i in range(ZR):
        for j in range(WOUT // 16):
            zbuf[i, pl.ds(j * 16, 16)] = jnp.zeros((16,), jnp.float32)

    gw = cid * 16 + sid
    ebase = gw * EPW

    def run_type(src_h, dst_h, xt_h, ad_h, c_h, out_h):
        def zloop(t, carry):
            pltpu.async_copy(zbuf, acc.at[pl.ds(sid * RPT + t * ZR, ZR)],
                             semg0)
            return carry

        lax.fori_loop(0, RPT // ZR, zloop, 0)

        def zdrain(t, carry):
            pltpu.make_async_copy(zbuf, acc.at[pl.ds(sid * RPT, ZR)],
                                  semg0).wait()
            return carry

        lax.fori_loop(0, RPT // ZR, zdrain, 0)
        pltpu.sync_copy(c_h, c_v)
        plsc.subcore_barrier()

        cv = c_v[...]

        def stage_gathers(eb1, q):
            pltpu.async_copy(ad_h.at[didx[q]], adr[q], semg[q])
            pltpu.async_copy(xt_h.at[sidx[q]], xm[q], semg[q])
            pltpu.async_copy(dst_h.at[pl.ds(eb1, K)], dsc[q], semg[q])

        def drain_gathers(b):
            pltpu.make_async_copy(ad_h.at[didx[b]], adr[b], semg[b]).wait()
            pltpu.make_async_copy(xt_h.at[sidx[b]], xm[b], semg[b]).wait()
            pltpu.make_async_copy(dst_h.at[pl.ds(ebase, K)], dsc[b],
                                  semg[b]).wait()

        def drain_scatter(b):
            pltpu.make_async_copy(xm[b], acc.at[dsc[b]], sems).wait()

        def drain_idx(q):
            pltpu.make_async_copy(src_h.at[pl.ds(ebase, K)], sidx[q],
                                  semi[q]).wait()
            pltpu.make_async_copy(dst_h.at[pl.ds(ebase, K)], didx[q],
                                  semi[q]).wait()

        for b in range(2):
            pltpu.sync_copy(src_h.at[pl.ds(ebase + b * K, K)], sidx[b])
            pltpu.sync_copy(dst_h.at[pl.ds(ebase + b * K, K)], didx[b])
            stage_gathers(ebase + b * K, b)

        def emit(ci, b):
            q = 1 - b
            drain_gathers(b)

            @pl.when(ci < NCH - 2)
            def _():
                eb2 = ebase + (ci + 2) * K
                pltpu.async_copy(src_h.at[pl.ds(eb2, K)], sidx[b], semi[b])
                pltpu.async_copy(dst_h.at[pl.ds(eb2, K)], didx[b], semi[b])

            @pl.when(ci >= 1)
            def _():
                drain_scatter(b)

            @pl.when((ci >= 1) & (ci < NCH - 1))
            def _():
                drain_idx(q)
                stage_gathers(ebase + (ci + 1) * K, q)

            eb = ebase + ci * K

            @plsc.parallel_loop(0, K, step=1, unroll=4)
            def _grp(k0):
                av = xm[b][k0, pl.ds(DH, 16)] + adr[b][k0]
                a = jnp.where(av >= 0.0, av, 0.2 * av)
                ev = jnp.exp(a - cv)
                ev = jnp.where(eb + k0 >= E, 0.0, ev)
                xm[b][k0, pl.ds(DH, 16)] = ev
                for h in range(H):
                    bc = ev.at[jnp.full((16,), h, jnp.int32)].get(
                        mode="promise_in_bounds")
                    xm[b][k0, pl.ds(h * 16, 16)] = (
                        xm[b][k0, pl.ds(h * 16, 16)] * bc)

            pltpu.async_copy(xm[b], acc.at[dsc[b]], sems, add=True)

        def pair(it, carry):
            emit(2 * it, 0)
            emit(2 * it + 1, 1)
            return carry

        lax.fori_loop(0, NCH // 2, pair, 0)
        drain_scatter(1)
        plsc.subcore_barrier()
        pltpu.sync_copy(acc.at[pl.ds(sid * RPT, RPT)],
                        out_h.at[cid, pl.ds(sid * RPT, RPT)])

    run_type(src1_h, dst1_h, xt1_h, ad1_h, c1_h, out1_h)
    run_type(src2_h, dst2_h, xt2_h, ad2_h, c2_h, out2_h)


_scratch_f32 = lambda shape: pltpu.VMEM(shape, jnp.float32)
_out_p = jax.ShapeDtypeStruct((2, N, WOUT), jnp.float32)
_sc_edge = functools.partial(
    pl.kernel,
    mesh=plsc.VectorSubcoreMesh(core_axis_name="c", subcore_axis_name="s"),
    compiler_params=pltpu.CompilerParams(use_tc_tiling_on_sc=False),
    out_type=[_out_p, _out_p],
    scratch_types=[
        pltpu.VMEM((K,), jnp.int32),
        pltpu.VMEM((K,), jnp.int32),
        pltpu.VMEM((K,), jnp.int32),
        pltpu.VMEM((K,), jnp.int32),
        pltpu.VMEM((K,), jnp.int32),
        pltpu.VMEM((K,), jnp.int32),
        _scratch_f32((K, HP)),
        _scratch_f32((K, WOUT)),
        _scratch_f32((K, HP)),
        _scratch_f32((K, WOUT)),
        _scratch_f32((ZR, WOUT)),
        _scratch_f32((16,)),
        pltpu.VMEM_SHARED((N, WOUT), jnp.float32),
        pltpu.SemaphoreType.DMA,
        pltpu.SemaphoreType.DMA,
        pltpu.SemaphoreType.DMA,
        pltpu.SemaphoreType.DMA,
        pltpu.SemaphoreType.DMA,
    ],
)(_sc_body)


def _tc2_body(p1, p2, Wk, bk, q2, Bx, Wo, bo, y_o, o1s, o2s, sacc):
    ph = pl.program_id(0)
    j = pl.program_id(1)

    @pl.when(ph == 0)
    def _():
        def one(p):
            ps = p[0] + p[1]
            denx = jnp.dot(ps[:, DH:WOUT], Bx[...],
                           preferred_element_type=jnp.float32)
            return jnp.maximum(ps[:, 0:DH] / (denx + 1e-16), 0.0)

        o1 = one(p1[...])
        o2 = one(p2[...])
        o1s[pl.ds(j * BLK, BLK), :] = o1
        o2s[pl.ds(j * BLK, BLK), :] = o2
        kp1 = jnp.tanh(jnp.dot(o1, Wk[...],
                               preferred_element_type=jnp.float32) + bk[...])
        kp2 = jnp.tanh(jnp.dot(o2, Wk[...],
                               preferred_element_type=jnp.float32) + bk[...])
        s1 = jnp.sum(kp1 * q2[...]) * (1.0 / N)
        s2 = jnp.sum(kp2 * q2[...]) * (1.0 / N)
        col = lax.broadcasted_iota(jnp.int32, (1, 8), 1)
        part = jnp.where(col == 0, s1, 0.0) + jnp.where(col == 1, s2, 0.0)

        @pl.when(j == 0)
        def _():
            sacc[...] = part

        @pl.when(j != 0)
        def _():
            sacc[...] = sacc[...] + part

    @pl.when(ph == 1)
    def _():
        sv = sacc[...]
        s1 = sv[0:1, 0:1]
        s2 = sv[0:1, 1:2]
        m = jnp.maximum(s1, s2)
        e1 = jnp.exp(s1 - m)
        e2 = jnp.exp(s2 - m)
        a1 = e1 / (e1 + e2)
        a2 = e2 / (e1 + e2)
        fused = (o1s[pl.ds(j * BLK, BLK), :] * a1 +
                 o2s[pl.ds(j * BLK, BLK), :] * a2)
        y_o[...] = jnp.dot(fused, Wo[...],
                           preferred_element_type=jnp.float32) + bo[...]


_tc2 = pl.pallas_call(
    _tc2_body,
    grid=(2, N // BLK),
    in_specs=[
        pl.BlockSpec((2, BLK, WOUT), lambda ph, j: (0, j * (1 - ph), 0)),
        pl.BlockSpec((2, BLK, WOUT), lambda ph, j: (0, j * (1 - ph), 0)),
        pl.BlockSpec((DH, DH), lambda ph, j: (0, 0)),
        pl.BlockSpec((1, DH), lambda ph, j: (0, 0)),
        pl.BlockSpec((1, DH), lambda ph, j: (0, 0)),
        pl.BlockSpec((HP, DH), lambda ph, j: (0, 0)),
        pl.BlockSpec((DH, DOUT), lambda ph, j: (0, 0)),
        pl.BlockSpec((1, DOUT), lambda ph, j: (0, 0)),
    ],
    out_specs=pl.BlockSpec((BLK, DOUT), lambda ph, j: (j, 0)),
    out_shape=jax.ShapeDtypeStruct((N, DOUT), jnp.float32),
    scratch_shapes=[
        pltpu.VMEM((N, DH), jnp.float32),
        pltpu.VMEM((N, DH), jnp.float32),
        pltpu.VMEM((1, 8), jnp.float32),
    ],
)


def _mkA(att):
    rows = jnp.arange(DH)[:, None]
    cols = jnp.arange(HP)[None, :]
    return jnp.where(cols == rows // D, att.reshape(-1)[:, None], 0.0)


def _leaky(v):
    return jnp.where(v >= 0.0, v, 0.2 * v)


def kernel(x_SUBJECT, x_ELECTRODE, edge_index_e2s, edge_index_s2s,
           W_proj_SUBJECT, b_proj_SUBJECT, W_proj_ELECTRODE, b_proj_ELECTRODE,
           att_src_e2s, att_dst_e2s, att_src_s2s, att_dst_s2s,
           W_k, b_k, q, W_out, b_out):
    A0 = _mkA(att_dst_e2s)
    A1 = _mkA(att_src_s2s)
    A2 = _mkA(att_dst_s2s)
    A3 = _mkA(att_src_e2s)

    xts, xte, t0, t2, mx = _tc1(
        x_SUBJECT, x_ELECTRODE, W_proj_SUBJECT, W_proj_ELECTRODE,
        b_proj_SUBJECT.reshape(1, DH), b_proj_ELECTRODE.reshape(1, DH),
        A0, A1, A2, A3)

    c_e2s = _leaky(mx[0, 48:64] + mx[0, 0:16])
    c_s2s = _leaky(mx[0, 16:32] + mx[0, 32:48])

    padv = jnp.broadcast_to((jnp.arange(EPAD - E, dtype=jnp.int32) % N)[None],
                            (2, EPAD - E))
    ei1 = jnp.concatenate([edge_index_e2s, padv], axis=1)
    ei2 = jnp.concatenate([edge_index_s2s, padv], axis=1)
    p1, p2 = _sc_edge(ei1[0], ei1[1], xte, t0, c_e2s,
                      ei2[0], ei2[1], xts, t2, c_s2s)

    rr = jnp.arange(HP)[:, None]
    cc = jnp.arange(DH)[None, :]
    Bx = jnp.where(rr == cc // D, 1.0, 0.0).astype(jnp.float32)

    return _tc2(p1, p2, W_k, b_k.reshape(1, DH), q.reshape(1, DH), Bx,
                W_out, b_out.reshape(1, DOUT))


# speedup vs baseline: 1.5010x; 1.1438x over previous
import functools

import jax
import jax.numpy as jnp
from jax import lax
from jax.experimental import pallas as pl
from jax.experimental.pallas import tpu as pltpu
from jax.experimental.pallas import tpu_sc as plsc

N = 10000
E = 160000
DIN = 128
DH = 128
H = 8
D = 16
HP = 16
DOUT = 8
WOUT = DH + HP

BLK = 2000
K = 112
EPW = 10080
EPAD = EPW * 16
NCH = EPW // K
RPT = N // 16
ZR = 5


def _tc1_body(xS, xE, WS, WE, bS, bE, A0, A1, A2, A3,
              xts_o, xte_o, t0_o, t2_o, mx_o):
    xs = jnp.dot(xS[...], WS[...], preferred_element_type=jnp.float32) + bS[...]
    xe = jnp.dot(xE[...], WE[...], preferred_element_type=jnp.float32) + bE[...]
    t0 = jnp.dot(xs, A0[...], preferred_element_type=jnp.float32)
    t1 = jnp.dot(xs, A1[...], preferred_element_type=jnp.float32)
    t2 = jnp.dot(xs, A2[...], preferred_element_type=jnp.float32)
    t3 = jnp.dot(xe, A3[...], preferred_element_type=jnp.float32)
    xts_o[...] = jnp.concatenate([xs, t1], axis=1)
    xte_o[...] = jnp.concatenate([xe, t3], axis=1)
    t0_o[...] = t0
    t2_o[...] = t2
    m = jnp.concatenate(
        [jnp.max(t0, axis=0, keepdims=True), jnp.max(t1, axis=0, keepdims=True),
         jnp.max(t2, axis=0, keepdims=True), jnp.max(t3, axis=0, keepdims=True)],
        axis=1)
    mb = jnp.broadcast_to(m, (8, 64))

    @pl.when(pl.program_id(0) == 0)
    def _():
        mx_o[...] = mb

    @pl.when(pl.program_id(0) != 0)
    def _():
        mx_o[...] = jnp.maximum(mx_o[...], mb)


_tbl = jax.ShapeDtypeStruct((N, HP), jnp.float32)
_tc1 = pl.pallas_call(
    _tc1_body,
    grid=(N // BLK,),
    in_specs=[
        pl.BlockSpec((BLK, DIN), lambda i: (i, 0)),
        pl.BlockSpec((BLK, DIN), lambda i: (i, 0)),
        pl.BlockSpec((DIN, DH), lambda i: (0, 0)),
        pl.BlockSpec((DIN, DH), lambda i: (0, 0)),
        pl.BlockSpec((1, DH), lambda i: (0, 0)),
        pl.BlockSpec((1, DH), lambda i: (0, 0)),
        pl.BlockSpec((DH, HP), lambda i: (0, 0)),
        pl.BlockSpec((DH, HP), lambda i: (0, 0)),
        pl.BlockSpec((DH, HP), lambda i: (0, 0)),
        pl.BlockSpec((DH, HP), lambda i: (0, 0)),
    ],
    out_specs=[
        pl.BlockSpec((BLK, WOUT), lambda i: (i, 0)),
        pl.BlockSpec((BLK, WOUT), lambda i: (i, 0)),
        pl.BlockSpec((BLK, HP), lambda i: (i, 0)),
        pl.BlockSpec((BLK, HP), lambda i: (i, 0)),
        pl.BlockSpec((8, 64), lambda i: (0, 0)),
    ],
    out_shape=[
        jax.ShapeDtypeStruct((N, WOUT), jnp.float32),
        jax.ShapeDtypeStruct((N, WOUT), jnp.float32),
        _tbl, _tbl,
        jax.ShapeDtypeStruct((8, 64), jnp.float32),
    ],
)


def _sc_body(src1_h, dst1_h, xt1_h, ad1_h, c1_h,
             src2_h, dst2_h, xt2_h, ad2_h, c2_h,
             out1_h, out2_h,
             sidx0, didx0, sidx1, didx1, dsc0, dsc1,
             adr0, xm0, adr1, xm1,
             zbuf, c_v, acc, semi0, semi1, semg0, semg1, sems):
    sidx = (sidx0, sidx1)
    didx = (didx0, didx1)
    dsc = (dsc0, dsc1)
    adr = (adr0, adr1)
    xm = (xm0, xm1)
    semi = (semi0, semi1)
    semg = (semg0, semg1)

    cid = lax.axis_index("c")
    sid = lax.axis_index("s")

    for i in range(ZR):
        for j in range(WOUT // 16):
            zbuf[i, pl.ds(j * 16, 16)] = jnp.zeros((16,), jnp.float32)

    ebase = sid * EPW

    def run_type(src_h, dst_h, xt_h, ad_h, c_h, out_h):
        def zloop(t, carry):
            pltpu.async_copy(zbuf, acc.at[pl.ds(sid * RPT + t * ZR, ZR)],
                             semg0)
            return carry

        lax.fori_loop(0, RPT // ZR, zloop, 0)

        def zdrain(t, carry):
            pltpu.make_async_copy(zbuf, acc.at[pl.ds(sid * RPT, ZR)],
                                  semg0).wait()
            return carry

        lax.fori_loop(0, RPT // ZR, zdrain, 0)
        pltpu.sync_copy(c_h, c_v)
        plsc.subcore_barrier()

        cv = c_v[...]

        def stage_gathers(eb1, q):
            pltpu.async_copy(ad_h.at[didx[q]], adr[q], semg[q])
            pltpu.async_copy(xt_h.at[sidx[q]], xm[q], semg[q])
            pltpu.async_copy(dst_h.at[pl.ds(eb1, K)], dsc[q], semg[q])

        def drain_gathers(b):
            pltpu.make_async_copy(ad_h.at[didx[b]], adr[b], semg[b]).wait()
            pltpu.make_async_copy(xt_h.at[sidx[b]], xm[b], semg[b]).wait()
            pltpu.make_async_copy(dst_h.at[pl.ds(ebase, K)], dsc[b],
                                  semg[b]).wait()

        def drain_scatter(b):
            pltpu.make_async_copy(xm[b], acc.at[dsc[b]], sems).wait()

        def drain_idx(q):
            pltpu.make_async_copy(src_h.at[pl.ds(ebase, K)], sidx[q],
                                  semi[q]).wait()
            pltpu.make_async_copy(dst_h.at[pl.ds(ebase, K)], didx[q],
                                  semi[q]).wait()

        for b in range(2):
            pltpu.sync_copy(src_h.at[pl.ds(ebase + b * K, K)], sidx[b])
            pltpu.sync_copy(dst_h.at[pl.ds(ebase + b * K, K)], didx[b])
            stage_gathers(ebase + b * K, b)

        def emit(ci, b):
            q = 1 - b
            drain_gathers(b)

            @pl.when(ci < NCH - 2)
            def _():
                eb2 = ebase + (ci + 2) * K
                pltpu.async_copy(src_h.at[pl.ds(eb2, K)], sidx[b], semi[b])
                pltpu.async_copy(dst_h.at[pl.ds(eb2, K)], didx[b], semi[b])

            @pl.when(ci >= 1)
            def _():
                drain_scatter(b)

            @pl.when((ci >= 1) & (ci < NCH - 1))
            def _():
                drain_idx(q)
                stage_gathers(ebase + (ci + 1) * K, q)

            eb = ebase + ci * K

            @plsc.parallel_loop(0, K, step=1, unroll=4)
            def _grp(k0):
                av = xm[b][k0, pl.ds(DH, 16)] + adr[b][k0]
                a = jnp.where(av >= 0.0, av, 0.2 * av)
                ev = jnp.exp(a - cv)
                ev = jnp.where(eb + k0 >= E, 0.0, ev)
                xm[b][k0, pl.ds(DH, 16)] = ev
                for h in range(H):
                    bc = ev.at[jnp.full((16,), h, jnp.int32)].get(
                        mode="promise_in_bounds")
                    xm[b][k0, pl.ds(h * 16, 16)] = (
                        xm[b][k0, pl.ds(h * 16, 16)] * bc)

            pltpu.async_copy(xm[b], acc.at[dsc[b]], sems, add=True)

        def pair(it, carry):
            emit(2 * it, 0)
            emit(2 * it + 1, 1)
            return carry

        lax.fori_loop(0, NCH // 2, pair, 0)
        drain_scatter(1)
        plsc.subcore_barrier()
        pltpu.sync_copy(acc.at[pl.ds(sid * RPT, RPT)],
                        out_h.at[pl.ds(sid * RPT, RPT)])

    @pl.when(cid == 0)
    def _():
        run_type(src1_h, dst1_h, xt1_h, ad1_h, c1_h, out1_h)

    @pl.when(cid == 1)
    def _():
        run_type(src2_h, dst2_h, xt2_h, ad2_h, c2_h, out2_h)


_scratch_f32 = lambda shape: pltpu.VMEM(shape, jnp.float32)
_out_p = jax.ShapeDtypeStruct((N, WOUT), jnp.float32)
_sc_edge = functools.partial(
    pl.kernel,
    mesh=plsc.VectorSubcoreMesh(core_axis_name="c", subcore_axis_name="s"),
    compiler_params=pltpu.CompilerParams(use_tc_tiling_on_sc=False),
    out_type=[_out_p, _out_p],
    scratch_types=[
        pltpu.VMEM((K,), jnp.int32),
        pltpu.VMEM((K,), jnp.int32),
        pltpu.VMEM((K,), jnp.int32),
        pltpu.VMEM((K,), jnp.int32),
        pltpu.VMEM((K,), jnp.int32),
        pltpu.VMEM((K,), jnp.int32),
        _scratch_f32((K, HP)),
        _scratch_f32((K, WOUT)),
        _scratch_f32((K, HP)),
        _scratch_f32((K, WOUT)),
        _scratch_f32((ZR, WOUT)),
        _scratch_f32((16,)),
        pltpu.VMEM_SHARED((N, WOUT), jnp.float32),
        pltpu.SemaphoreType.DMA,
        pltpu.SemaphoreType.DMA,
        pltpu.SemaphoreType.DMA,
        pltpu.SemaphoreType.DMA,
        pltpu.SemaphoreType.DMA,
    ],
)(_sc_body)


def _tc2_body(p1, p2, Wk, bk, q2, Bx, Wo, bo, y_o, o1s, o2s, sacc):
    ph = pl.program_id(0)
    j = pl.program_id(1)

    @pl.when(ph == 0)
    def _():
        def one(p):
            ps = p[...]
            denx = jnp.dot(ps[:, DH:WOUT], Bx[...],
                           preferred_element_type=jnp.float32)
            return jnp.maximum(ps[:, 0:DH] / (denx + 1e-16), 0.0)

        o1 = one(p1)
        o2 = one(p2)
        o1s[pl.ds(j * BLK, BLK), :] = o1
        o2s[pl.ds(j * BLK, BLK), :] = o2
        kp1 = jnp.tanh(jnp.dot(o1, Wk[...],
                               preferred_element_type=jnp.float32) + bk[...])
        kp2 = jnp.tanh(jnp.dot(o2, Wk[...],
                               preferred_element_type=jnp.float32) + bk[...])
        s1 = jnp.sum(kp1 * q2[...]) * (1.0 / N)
        s2 = jnp.sum(kp2 * q2[...]) * (1.0 / N)
        col = lax.broadcasted_iota(jnp.int32, (1, 8), 1)
        part = jnp.where(col == 0, s1, 0.0) + jnp.where(col == 1, s2, 0.0)

        @pl.when(j == 0)
        def _():
            sacc[...] = part

        @pl.when(j != 0)
        def _():
            sacc[...] = sacc[...] + part

    @pl.when(ph == 1)
    def _():
        sv = sacc[...]
        s1 = sv[0:1, 0:1]
        s2 = sv[0:1, 1:2]
        m = jnp.maximum(s1, s2)
        e1 = jnp.exp(s1 - m)
        e2 = jnp.exp(s2 - m)
        a1 = e1 / (e1 + e2)
        a2 = e2 / (e1 + e2)
        fused = (o1s[pl.ds(j * BLK, BLK), :] * a1 +
                 o2s[pl.ds(j * BLK, BLK), :] * a2)
        y_o[...] = jnp.dot(fused, Wo[...],
                           preferred_element_type=jnp.float32) + bo[...]


_tc2 = pl.pallas_call(
    _tc2_body,
    grid=(2, N // BLK),
    in_specs=[
        pl.BlockSpec((BLK, WOUT), lambda ph, j: (j * (1 - ph), 0)),
        pl.BlockSpec((BLK, WOUT), lambda ph, j: (j * (1 - ph), 0)),
        pl.BlockSpec((DH, DH), lambda ph, j: (0, 0)),
        pl.BlockSpec((1, DH), lambda ph, j: (0, 0)),
        pl.BlockSpec((1, DH), lambda ph, j: (0, 0)),
        pl.BlockSpec((HP, DH), lambda ph, j: (0, 0)),
        pl.BlockSpec((DH, DOUT), lambda ph, j: (0, 0)),
        pl.BlockSpec((1, DOUT), lambda ph, j: (0, 0)),
    ],
    out_specs=pl.BlockSpec((BLK, DOUT), lambda ph, j: (j, 0)),
    out_shape=jax.ShapeDtypeStruct((N, DOUT), jnp.float32),
    scratch_shapes=[
        pltpu.VMEM((N, DH), jnp.float32),
        pltpu.VMEM((N, DH), jnp.float32),
        pltpu.VMEM((1, 8), jnp.float32),
    ],
)


def _mkA(att):
    rows = jnp.arange(DH)[:, None]
    cols = jnp.arange(HP)[None, :]
    return jnp.where(cols == rows // D, att.reshape(-1)[:, None], 0.0)


def _leaky(v):
    return jnp.where(v >= 0.0, v, 0.2 * v)


def kernel(x_SUBJECT, x_ELECTRODE, edge_index_e2s, edge_index_s2s,
           W_proj_SUBJECT, b_proj_SUBJECT, W_proj_ELECTRODE, b_proj_ELECTRODE,
           att_src_e2s, att_dst_e2s, att_src_s2s, att_dst_s2s,
           W_k, b_k, q, W_out, b_out):
    A0 = _mkA(att_dst_e2s)
    A1 = _mkA(att_src_s2s)
    A2 = _mkA(att_dst_s2s)
    A3 = _mkA(att_src_e2s)

    xts, xte, t0, t2, mx = _tc1(
        x_SUBJECT, x_ELECTRODE, W_proj_SUBJECT, W_proj_ELECTRODE,
        b_proj_SUBJECT.reshape(1, DH), b_proj_ELECTRODE.reshape(1, DH),
        A0, A1, A2, A3)

    c_e2s = _leaky(mx[0, 48:64] + mx[0, 0:16])
    c_s2s = _leaky(mx[0, 16:32] + mx[0, 32:48])

    padv = jnp.broadcast_to((jnp.arange(EPAD - E, dtype=jnp.int32) % N)[None],
                            (2, EPAD - E))
    ei1 = jnp.concatenate([edge_index_e2s, padv], axis=1)
    ei2 = jnp.concatenate([edge_index_s2s, padv], axis=1)
    p1, p2 = _sc_edge(ei1[0], ei1[1], xte, t0, c_e2s,
                      ei2[0], ei2[1], xts, t2, c_s2s)

    rr = jnp.arange(HP)[:, None]
    cc = jnp.arange(DH)[None, :]
    Bx = jnp.where(rr == cc // D, 1.0, 0.0).astype(jnp.float32)

    return _tc2(p1, p2, W_k, b_k.reshape(1, DH), q.reshape(1, DH), Bx,
                W_out, b_out.reshape(1, DOUT))
